# baseline (speedup 1.0000x reference)
import functools

import jax
import jax.numpy as jnp
from jax import lax
from jax.experimental import pallas as pl
from jax.experimental.pallas import tpu as pltpu
from jax.experimental.pallas import tpu_sc as plsc

_NC = 2
_NS = 16
_L = 16
_NW = _NC * _NS
_CHUNK = 128
_CW = 98304


def _dense_scores(tableT, wcb):
    D, V = tableT.shape

    def body(t_ref, w_ref, s_ref):
        x = t_ref[...]
        w = w_ref[:, 0:1]
        s_ref[...] = jnp.sum(x * w, axis=0)

    return pl.pallas_call(
        body,
        grid=(pl.cdiv(V, _CW),),
        compiler_params=pltpu.CompilerParams(
            dimension_semantics=("parallel",)),
        in_specs=[
            pl.BlockSpec((D, _CW), lambda i: (0, i)),
            pl.BlockSpec((D, 128), lambda i: (0, 0)),
        ],
        out_specs=pl.BlockSpec((_CW,), lambda i: (i,)),
        out_shape=jax.ShapeDtypeStruct((V,), jnp.float32),
    )(tableT, wcb)


@functools.lru_cache(maxsize=None)
def _build_pick(B, V):
    assert B % (_NW * _CHUNK) == 0
    bpw = B // _NW
    nch = bpw // _CHUNK

    mesh = plsc.VectorSubcoreMesh(core_axis_name="c", subcore_axis_name="s")

    @functools.partial(
        pl.kernel,
        mesh=mesh,
        compiler_params=pltpu.CompilerParams(
            needs_layout_passes=False, use_tc_tiling_on_sc=False),
        out_type=jax.ShapeDtypeStruct((B,), jnp.float32),
        scratch_types=[
            pltpu.VMEM((nch, _CHUNK), jnp.int32),
            pltpu.VMEM((bpw,), jnp.float32),
            pltpu.VMEM((_L,), jnp.float32),
            pltpu.SemaphoreType.DMA,
        ],
    )
    def pick(idx_hbm, bias_hbm, s_hbm, out_hbm, idx_v, val_v, bias_v, sem):
        wid = lax.axis_index("s") * _NC + lax.axis_index("c")
        base = wid * bpw
        pltpu.sync_copy(idx_hbm.at[wid], idx_v)
        pltpu.sync_copy(bias_hbm, bias_v)
        copies = [
            pltpu.async_copy(
                s_hbm.at[idx_v.at[j]],
                val_v.at[pl.ds(j * _CHUNK, _CHUNK)], sem)
            for j in range(nch)
        ]
        for h in copies:
            h.wait()
        bias = bias_v[...]
        for j in range(bpw // _L):
            o = j * _L
            val_v[pl.ds(o, _L)] = val_v[pl.ds(o, _L)] + bias
        pltpu.sync_copy(val_v, out_hbm.at[pl.ds(base, bpw)])

    return pick


def kernel(x_movie, x_user, movie_table, fc_w, fc_b):
    B = x_movie.shape[0]
    V, D = movie_table.shape
    wc = fc_w[0, :D] + fc_w[0, D:]
    wcb = jnp.broadcast_to(wc[:, None], (D, 128))
    bias = jnp.broadcast_to(fc_b, (_L,)).astype(jnp.float32)
    idx = x_movie.astype(jnp.int32).reshape(_NW, B // (_NW * _CHUNK), _CHUNK)
    s = _dense_scores(movie_table.T, wcb)
    out = _build_pick(B, V)(idx, bias, s)
    return out.reshape(B, 1)


# speedup vs baseline: 1.0139x; 1.0139x over previous
import functools

import jax
import jax.numpy as jnp
from jax import lax
from jax.experimental import pallas as pl
from jax.experimental.pallas import tpu as pltpu
from jax.experimental.pallas import tpu_sc as plsc

_NC = 2
_NS = 16
_L = 16
_NW = _NC * _NS
_CHUNK = 128
_CW = 65536


def _dense_scores(tableT, wcb):
    D, V = tableT.shape

    def body(t_ref, w_ref, s_ref):
        x = t_ref[...]
        w = w_ref[:, 0:1]
        s_ref[...] = jnp.sum(x * w, axis=0)

    return pl.pallas_call(
        body,
        grid=(pl.cdiv(V, _CW),),
        compiler_params=pltpu.CompilerParams(
            dimension_semantics=("parallel",)),
        in_specs=[
            pl.BlockSpec((D, _CW), lambda i: (0, i)),
            pl.BlockSpec((D, 128), lambda i: (0, 0)),
        ],
        out_specs=pl.BlockSpec((_CW,), lambda i: (i,)),
        out_shape=jax.ShapeDtypeStruct((V,), jnp.float32),
    )(tableT, wcb)


@functools.lru_cache(maxsize=None)
def _build_pick(B, V):
    assert B % (_NW * _CHUNK) == 0
    bpw = B // _NW
    nch = bpw // _CHUNK

    mesh = plsc.VectorSubcoreMesh(core_axis_name="c", subcore_axis_name="s")

    @functools.partial(
        pl.kernel,
        mesh=mesh,
        compiler_params=pltpu.CompilerParams(
            needs_layout_passes=False, use_tc_tiling_on_sc=False),
        out_type=jax.ShapeDtypeStruct((B,), jnp.float32),
        scratch_types=[
            pltpu.VMEM((nch, _CHUNK), jnp.int32),
            pltpu.VMEM((bpw,), jnp.float32),
            pltpu.VMEM((_L,), jnp.float32),
            pltpu.SemaphoreType.DMA,
        ],
    )
    def pick(idx_hbm, bias_hbm, s_hbm, out_hbm, idx_v, val_v, bias_v, sem):
        wid = lax.axis_index("s") * _NC + lax.axis_index("c")
        base = wid * bpw
        pltpu.sync_copy(idx_hbm.at[wid], idx_v)
        pltpu.sync_copy(bias_hbm, bias_v)
        copies = [
            pltpu.async_copy(
                s_hbm.at[idx_v.at[j]],
                val_v.at[pl.ds(j * _CHUNK, _CHUNK)], sem)
            for j in range(nch)
        ]
        for h in copies:
            h.wait()
        bias = bias_v[...]
        for j in range(bpw // _L):
            o = j * _L
            val_v[pl.ds(o, _L)] = val_v[pl.ds(o, _L)] + bias
        pltpu.sync_copy(val_v, out_hbm.at[pl.ds(base, bpw)])

    return pick


def kernel(x_movie, x_user, movie_table, fc_w, fc_b):
    B = x_movie.shape[0]
    V, D = movie_table.shape
    wc = fc_w[0, :D] + fc_w[0, D:]
    wcb = jnp.broadcast_to(wc[:, None], (D, 128))
    bias = jnp.broadcast_to(fc_b, (_L,)).astype(jnp.float32)
    idx = x_movie.astype(jnp.int32).reshape(_NW, B // (_NW * _CHUNK), _CHUNK)
    s = _dense_scores(movie_table.T, wcb)
    out = _build_pick(B, V)(idx, bias, s)
    return out.reshape(B, 1)
